# Initial kernel scaffold; baseline (speedup 1.0000x reference)
import jax
import jax.numpy as jnp
from jax.experimental import pallas as pl


def kernel(x, edge_index, edge_attr, W1, b1, g1, be1, W2, b2, g2, be2, Wc1, bc1, Wc2, bc2):
    raise NotImplementedError("write your pallas kernel here")


# speedup vs baseline: 9.6304x; 9.6304x over previous
import functools

import jax
import jax.numpy as jnp
from jax import lax
from jax.experimental import pallas as pl
from jax.experimental.pallas import tpu as pltpu
from jax.experimental.pallas import tpu_sc as plsc

NC, NS, L = 2, 16, 16
NW = NC * NS


def _make_deg_kernel(E, Npad):
    Ept = E // NW
    C = 80
    assert Ept % C == 0 and C % 8 == 0
    Nt = Npad // NS
    assert Nt % L == 0 and (Nt % 8) == 0
    mesh = plsc.VectorSubcoreMesh(core_axis_name="c", subcore_axis_name="s")

    @functools.partial(
        pl.kernel,
        out_type=jax.ShapeDtypeStruct((NC, Npad), jnp.float32),
        mesh=mesh,
        scratch_types=[
            pltpu.VMEM((C,), jnp.int32),
            pltpu.VMEM((C,), jnp.float32),
            pltpu.VMEM((Nt,), jnp.float32),
            pltpu.VMEM_SHARED((Npad,), jnp.float32),
        ],
    )
    def deg_kernel(col_hbm, ew_hbm, degp_hbm, colb, ewb, bounce, degsh):
        cid = lax.axis_index("c")
        sid = lax.axis_index("s")
        wid = sid * NC + cid
        nbase = sid * Nt

        zero = jnp.zeros((L,), jnp.float32)

        def _z(i, carry):
            bounce[pl.ds(i * L, L)] = zero
            return carry

        lax.fori_loop(0, Nt // L, _z, 0)
        pltpu.sync_copy(bounce, degsh.at[pl.ds(nbase, Nt)])
        plsc.subcore_barrier()

        ebase = wid * Ept

        def _chunk(j, carry):
            base = ebase + j * C
            pltpu.sync_copy(col_hbm.at[pl.ds(base, C)], colb)
            pltpu.sync_copy(ew_hbm.at[pl.ds(base, C)], ewb)
            pltpu.sync_copy(ewb, degsh.at[colb], add=True)
            return carry

        lax.fori_loop(0, Ept // C, _chunk, 0)

        plsc.subcore_barrier()
        pltpu.sync_copy(degsh.at[pl.ds(nbase, Nt)], bounce)
        pltpu.sync_copy(bounce, degp_hbm.at[cid, pl.ds(nbase, Nt)])

    return deg_kernel


def _make_msg_kernel(Npad, E, H):
    Ept = E // NW
    C = 80
    assert Ept % C == 0 and C % 8 == 0
    Nt = Npad // NS
    mesh = plsc.VectorSubcoreMesh(core_axis_name="c", subcore_axis_name="s")

    @functools.partial(
        pl.kernel,
        out_type=jax.ShapeDtypeStruct((NC, Npad, H), jnp.float32),
        mesh=mesh,
        compiler_params=pltpu.CompilerParams(use_tc_tiling_on_sc=False),
        scratch_types=[
            pltpu.VMEM((C,), jnp.int32),
            pltpu.VMEM((C,), jnp.int32),
            pltpu.VMEM((C + L,), jnp.float32),
            pltpu.VMEM((C, H), jnp.float32),
            pltpu.VMEM((Nt, H), jnp.float32),
            pltpu.VMEM_SHARED((Npad, H), jnp.float32),
            pltpu.SemaphoreType.DMA,
        ],
    )
    def msg_kernel(y_hbm, row_hbm, col_hbm, ew_hbm, sp_hbm,
                   rowb, colb, ewb, rows, bounce, ssh, sem):
        cid = lax.axis_index("c")
        sid = lax.axis_index("s")
        wid = sid * NC + cid
        nbase = sid * Nt

        zero = jnp.zeros((L,), jnp.float32)
        KH = H // L

        def _z(t, carry):
            i = t // KH
            k = t % KH
            bounce[i, pl.ds(k * L, L)] = zero
            return carry

        lax.fori_loop(0, Nt * KH, _z, 0)
        pltpu.sync_copy(bounce, ssh.at[pl.ds(nbase, Nt)])
        plsc.subcore_barrier()

        ebase = wid * Ept

        def _chunk(j, carry):
            base = ebase + j * C
            pltpu.sync_copy(row_hbm.at[pl.ds(base, C)], rowb)
            pltpu.sync_copy(col_hbm.at[pl.ds(base, C)], colb)
            pltpu.sync_copy(ew_hbm.at[pl.ds(base, C)], ewb.at[pl.ds(0, C)])
            pltpu.async_copy(y_hbm.at[rowb], rows, sem).wait()

            def _scale(e, c2):
                ev = ewb[pl.ds(e, L)][0]
                for k in range(KH):
                    rows[e, pl.ds(k * L, L)] = rows[e, pl.ds(k * L, L)] * ev
                return c2

            lax.fori_loop(0, C, _scale, 0)
            pltpu.sync_copy(rows, ssh.at[colb], add=True)
            return carry

        lax.fori_loop(0, Ept // C, _chunk, 0)

        plsc.subcore_barrier()
        pltpu.sync_copy(ssh.at[pl.ds(nbase, Nt)], bounce)
        pltpu.sync_copy(bounce, sp_hbm.at[cid, pl.ds(nbase, Nt)])

    return msg_kernel


def _tc1(x, W1, deg0, deg1, Npad, BN=640):
    N, F = x.shape
    H = W1.shape[1]

    def body(x_ref, w_ref, d0_ref, d1_ref, y_ref, dv_ref):
        deg = d0_ref[...] + d1_ref[...] + 1.0
        dinv = lax.rsqrt(deg)
        xw = jnp.dot(x_ref[...], w_ref[...], preferred_element_type=jnp.float32)
        y_ref[...] = xw * dinv
        dv_ref[...] = dinv

    return pl.pallas_call(
        body,
        grid=(Npad // BN,),
        in_specs=[
            pl.BlockSpec((BN, F), lambda i: (i, 0)),
            pl.BlockSpec((F, H), lambda i: (0, 0)),
            pl.BlockSpec((BN, 1), lambda i: (i, 0)),
            pl.BlockSpec((BN, 1), lambda i: (i, 0)),
        ],
        out_specs=[
            pl.BlockSpec((BN, H), lambda i: (i, 0)),
            pl.BlockSpec((BN, 1), lambda i: (i, 0)),
        ],
        out_shape=[
            jax.ShapeDtypeStruct((Npad, H), jnp.float32),
            jax.ShapeDtypeStruct((Npad, 1), jnp.float32),
        ],
    )(x, W1, deg0, deg1)


def _tc2(sp0, sp1, y1, dinv, a1, c1, W2, BN=640):
    Npad, H = y1.shape

    def body(s0_ref, s1_ref, y1_ref, dv_ref, a_ref, c_ref, w_ref, y2_ref):
        s = s0_ref[...] + s1_ref[...] + y1_ref[...]
        m = s * dv_ref[...]
        h = jnp.maximum(m * a_ref[...] + c_ref[...], 0.0)
        y2_ref[...] = jnp.dot(h, w_ref[...], preferred_element_type=jnp.float32) * dv_ref[...]

    return pl.pallas_call(
        body,
        grid=(Npad // BN,),
        in_specs=[
            pl.BlockSpec((BN, H), lambda i: (i, 0)),
            pl.BlockSpec((BN, H), lambda i: (i, 0)),
            pl.BlockSpec((BN, H), lambda i: (i, 0)),
            pl.BlockSpec((BN, 1), lambda i: (i, 0)),
            pl.BlockSpec((1, H), lambda i: (0, 0)),
            pl.BlockSpec((1, H), lambda i: (0, 0)),
            pl.BlockSpec((H, H), lambda i: (0, 0)),
        ],
        out_specs=pl.BlockSpec((BN, H), lambda i: (i, 0)),
        out_shape=jax.ShapeDtypeStruct((Npad, H), jnp.float32),
    )(sp0, sp1, y1, dinv, a1, c1, W2)


def _tc3(sp0, sp1, y2, dinv, a2, c2, Wc1, bc1, Wc2, bc2, BN=640):
    Npad, H = y2.shape
    C1 = Wc1.shape[1]

    def body(s0_ref, s1_ref, y2_ref, dv_ref, a_ref, c_ref, wc1_ref, bc1_ref,
             wc2_ref, bc2_ref, o_ref):
        s = s0_ref[...] + s1_ref[...] + y2_ref[...]
        m = s * dv_ref[...]
        h = jnp.maximum(m * a_ref[...] + c_ref[...], 0.0)
        z = jnp.maximum(
            jnp.dot(h, wc1_ref[...], preferred_element_type=jnp.float32)
            + bc1_ref[...], 0.0)
        o = jnp.dot(z, wc2_ref[...], preferred_element_type=jnp.float32) + bc2_ref[...]
        o_ref[...] = jax.nn.sigmoid(o)

    return pl.pallas_call(
        body,
        grid=(Npad // BN,),
        in_specs=[
            pl.BlockSpec((BN, H), lambda i: (i, 0)),
            pl.BlockSpec((BN, H), lambda i: (i, 0)),
            pl.BlockSpec((BN, H), lambda i: (i, 0)),
            pl.BlockSpec((BN, 1), lambda i: (i, 0)),
            pl.BlockSpec((1, H), lambda i: (0, 0)),
            pl.BlockSpec((1, H), lambda i: (0, 0)),
            pl.BlockSpec((H, C1), lambda i: (0, 0)),
            pl.BlockSpec((1, C1), lambda i: (0, 0)),
            pl.BlockSpec((C1, 1), lambda i: (0, 0)),
            pl.BlockSpec((1, 1), lambda i: (0, 0)),
        ],
        out_specs=pl.BlockSpec((BN, 1), lambda i: (i, 0)),
        out_shape=jax.ShapeDtypeStruct((Npad, 1), jnp.float32),
    )(sp0, sp1, y2, dinv, a2, c2, Wc1, bc1, Wc2, bc2)


def kernel(x, edge_index, edge_attr, W1, b1, g1, be1, W2, b2, g2, be2,
           Wc1, bc1, Wc2, bc2):
    N, F = x.shape
    H = W1.shape[1]
    C1 = Wc1.shape[1]
    E = edge_attr.shape[0]
    Npad = ((N + NS * L - 1) // (NS * L)) * (NS * L)

    row = edge_index[0].astype(jnp.int32)
    col = edge_index[1].astype(jnp.int32)
    ew = edge_attr.astype(jnp.float32)

    deg_call = _make_deg_kernel(E, Npad)
    msg_call = _make_msg_kernel(Npad, E, H)

    degp = deg_call(col, ew)
    y1, dinv = _tc1(x, W1, degp[0].reshape(Npad, 1), degp[1].reshape(Npad, 1),
                    Npad)

    isq = float(1.0 + 1e-5) ** -0.5
    a1 = (g1 * isq).reshape(1, H)
    c1 = (g1 * isq * b1 + be1).reshape(1, H)
    a2 = (g2 * isq).reshape(1, H)
    c2 = (g2 * isq * b2 + be2).reshape(1, H)

    sp1 = msg_call(y1, row, col, ew)
    y2 = _tc2(sp1[0], sp1[1], y1, dinv, a1, c1, W2)
    sp2 = msg_call(y2, row, col, ew)
    out = _tc3(sp2[0], sp2[1], y2, dinv, a2, c2, Wc1, bc1.reshape(1, C1),
               Wc2, bc2.reshape(1, 1))
    return out.reshape(Npad)[:N]


# speedup vs baseline: 18.4748x; 1.9184x over previous
import functools

import jax
import jax.numpy as jnp
from jax import lax
from jax.experimental import pallas as pl
from jax.experimental.pallas import tpu as pltpu
from jax.experimental.pallas import tpu_sc as plsc

NC, NS, L = 2, 16, 16
NW = NC * NS


def _make_deg_kernel(E, Npad, C):
    Ept = E // NW
    RPT = Ept // C
    K = 25
    assert RPT % K == 0
    Nt = Npad // NS
    assert Nt % L == 0 and (Nt % 8) == 0
    mesh = plsc.VectorSubcoreMesh(core_axis_name="c", subcore_axis_name="s")

    @functools.partial(
        pl.kernel,
        out_type=jax.ShapeDtypeStruct((NC, Npad), jnp.float32),
        mesh=mesh,
        compiler_params=pltpu.CompilerParams(use_tc_tiling_on_sc=False),
        scratch_types=[
            pltpu.VMEM((K, C), jnp.int32),
            pltpu.VMEM((K, C), jnp.float32),
            pltpu.VMEM((Nt,), jnp.float32),
            pltpu.VMEM_SHARED((Npad,), jnp.float32),
            pltpu.SemaphoreType.DMA,
        ],
    )
    def deg_kernel(col_hbm, ew_hbm, degp_hbm, colb, ewb, bounce, degsh, sem):
        cid = lax.axis_index("c")
        sid = lax.axis_index("s")
        wid = sid * NC + cid
        nbase = sid * Nt

        zero = jnp.zeros((L,), jnp.float32)

        def _z(i, carry):
            bounce[pl.ds(i * L, L)] = zero
            return carry

        lax.fori_loop(0, Nt // L, _z, 0)
        pltpu.sync_copy(bounce, degsh.at[pl.ds(nbase, Nt)])
        plsc.subcore_barrier()

        rbase = wid * RPT

        def _group(g, carry):
            r0 = rbase + g * K
            pltpu.sync_copy(col_hbm.at[pl.ds(r0, K)], colb)
            pltpu.sync_copy(ew_hbm.at[pl.ds(r0, K)], ewb)
            descs = []
            for q in range(K):
                descs.append(pltpu.async_copy(
                    ewb.at[q], degsh.at[colb.at[q]], sem, add=True))
            for d in descs:
                d.wait()
            return carry

        lax.fori_loop(0, RPT // K, _group, 0)

        plsc.subcore_barrier()
        pltpu.sync_copy(degsh.at[pl.ds(nbase, Nt)], bounce)
        pltpu.sync_copy(bounce, degp_hbm.at[cid, pl.ds(nbase, Nt)])

    return deg_kernel


def _make_msg_kernel(Npad, E, H, C):
    Ept = E // NW
    RPT = Ept // C
    K = 5
    assert RPT % K == 0
    Nt = Npad // NS
    mesh = plsc.VectorSubcoreMesh(core_axis_name="c", subcore_axis_name="s")

    @functools.partial(
        pl.kernel,
        out_type=jax.ShapeDtypeStruct((NC, Npad, H), jnp.float32),
        mesh=mesh,
        compiler_params=pltpu.CompilerParams(use_tc_tiling_on_sc=False),
        scratch_types=[
            pltpu.VMEM((K, C), jnp.int32),
            pltpu.VMEM((K, C), jnp.int32),
            pltpu.VMEM((K + 1, C), jnp.float32),
            pltpu.VMEM((K * C, H), jnp.float32),
            pltpu.VMEM((Nt, H), jnp.float32),
            pltpu.VMEM_SHARED((Npad, H), jnp.float32),
            pltpu.SemaphoreType.DMA,
            pltpu.SemaphoreType.DMA,
        ],
    )
    def msg_kernel(y_hbm, row_hbm, col_hbm, ew_hbm, sp_hbm,
                   rowb, colb, ewb, rows, bounce, ssh, semg, sems):
        cid = lax.axis_index("c")
        sid = lax.axis_index("s")
        wid = sid * NC + cid
        nbase = sid * Nt

        zero = jnp.zeros((L,), jnp.float32)
        KH = H // L

        def _z(t, carry):
            i = t // KH
            k = t % KH
            bounce[i, pl.ds(k * L, L)] = zero
            return carry

        lax.fori_loop(0, Nt * KH, _z, 0)
        pltpu.sync_copy(bounce, ssh.at[pl.ds(nbase, Nt)])
        plsc.subcore_barrier()

        rbase = wid * RPT

        def _group(g, carry):
            r0 = rbase + g * K
            pltpu.sync_copy(row_hbm.at[pl.ds(r0, K)], rowb)
            pltpu.sync_copy(col_hbm.at[pl.ds(r0, K)], colb)
            pltpu.sync_copy(ew_hbm.at[pl.ds(r0, K)], ewb.at[pl.ds(0, K)])
            gd = []
            for q in range(K):
                gd.append(pltpu.async_copy(
                    y_hbm.at[rowb.at[q]], rows.at[pl.ds(q * C, C)], semg))
            for d in gd:
                d.wait()

            for q in range(K):
                def _scale(e, c2, q=q):
                    ev = ewb[q, pl.ds(e, L)][0]
                    for k in range(KH):
                        rows[q * C + e, pl.ds(k * L, L)] = (
                            rows[q * C + e, pl.ds(k * L, L)] * ev)
                    return c2
                lax.fori_loop(0, C, _scale, 0)

            sd = []
            for q in range(K):
                sd.append(pltpu.async_copy(
                    rows.at[pl.ds(q * C, C)], ssh.at[colb.at[q]], sems,
                    add=True))
            for d in sd:
                d.wait()
            return carry

        lax.fori_loop(0, RPT // K, _group, 0)

        plsc.subcore_barrier()
        pltpu.sync_copy(ssh.at[pl.ds(nbase, Nt)], bounce)
        pltpu.sync_copy(bounce, sp_hbm.at[cid, pl.ds(nbase, Nt)])

    return msg_kernel


def _tc1(x, W1, deg0, deg1, Npad, BN=640):
    N, F = x.shape
    H = W1.shape[1]

    def body(x_ref, w_ref, d0_ref, d1_ref, y_ref, dv_ref):
        deg = d0_ref[...] + d1_ref[...] + 1.0
        dinv = lax.rsqrt(deg)
        xw = jnp.dot(x_ref[...], w_ref[...], preferred_element_type=jnp.float32)
        y_ref[...] = xw * dinv
        dv_ref[...] = dinv

    return pl.pallas_call(
        body,
        grid=(Npad // BN,),
        in_specs=[
            pl.BlockSpec((BN, F), lambda i: (i, 0)),
            pl.BlockSpec((F, H), lambda i: (0, 0)),
            pl.BlockSpec((BN, 1), lambda i: (i, 0)),
            pl.BlockSpec((BN, 1), lambda i: (i, 0)),
        ],
        out_specs=[
            pl.BlockSpec((BN, H), lambda i: (i, 0)),
            pl.BlockSpec((BN, 1), lambda i: (i, 0)),
        ],
        out_shape=[
            jax.ShapeDtypeStruct((Npad, H), jnp.float32),
            jax.ShapeDtypeStruct((Npad, 1), jnp.float32),
        ],
    )(x, W1, deg0, deg1)


def _tc2(sp0, sp1, y1, dinv, a1, c1, W2, BN=640):
    Npad, H = y1.shape

    def body(s0_ref, s1_ref, y1_ref, dv_ref, a_ref, c_ref, w_ref, y2_ref):
        s = s0_ref[...] + s1_ref[...] + y1_ref[...]
        m = s * dv_ref[...]
        h = jnp.maximum(m * a_ref[...] + c_ref[...], 0.0)
        y2_ref[...] = jnp.dot(h, w_ref[...], preferred_element_type=jnp.float32) * dv_ref[...]

    return pl.pallas_call(
        body,
        grid=(Npad // BN,),
        in_specs=[
            pl.BlockSpec((BN, H), lambda i: (i, 0)),
            pl.BlockSpec((BN, H), lambda i: (i, 0)),
            pl.BlockSpec((BN, H), lambda i: (i, 0)),
            pl.BlockSpec((BN, 1), lambda i: (i, 0)),
            pl.BlockSpec((1, H), lambda i: (0, 0)),
            pl.BlockSpec((1, H), lambda i: (0, 0)),
            pl.BlockSpec((H, H), lambda i: (0, 0)),
        ],
        out_specs=pl.BlockSpec((BN, H), lambda i: (i, 0)),
        out_shape=jax.ShapeDtypeStruct((Npad, H), jnp.float32),
    )(sp0, sp1, y1, dinv, a1, c1, W2)


def _tc3(sp0, sp1, y2, dinv, a2, c2, Wc1, bc1, Wc2, bc2, BN=640):
    Npad, H = y2.shape
    C1 = Wc1.shape[1]

    def body(s0_ref, s1_ref, y2_ref, dv_ref, a_ref, c_ref, wc1_ref, bc1_ref,
             wc2_ref, bc2_ref, o_ref):
        s = s0_ref[...] + s1_ref[...] + y2_ref[...]
        m = s * dv_ref[...]
        h = jnp.maximum(m * a_ref[...] + c_ref[...], 0.0)
        z = jnp.maximum(
            jnp.dot(h, wc1_ref[...], preferred_element_type=jnp.float32)
            + bc1_ref[...], 0.0)
        o = jnp.dot(z, wc2_ref[...], preferred_element_type=jnp.float32) + bc2_ref[...]
        o_ref[...] = jax.nn.sigmoid(o)

    return pl.pallas_call(
        body,
        grid=(Npad // BN,),
        in_specs=[
            pl.BlockSpec((BN, H), lambda i: (i, 0)),
            pl.BlockSpec((BN, H), lambda i: (i, 0)),
            pl.BlockSpec((BN, H), lambda i: (i, 0)),
            pl.BlockSpec((BN, 1), lambda i: (i, 0)),
            pl.BlockSpec((1, H), lambda i: (0, 0)),
            pl.BlockSpec((1, H), lambda i: (0, 0)),
            pl.BlockSpec((H, C1), lambda i: (0, 0)),
            pl.BlockSpec((1, C1), lambda i: (0, 0)),
            pl.BlockSpec((C1, 1), lambda i: (0, 0)),
            pl.BlockSpec((1, 1), lambda i: (0, 0)),
        ],
        out_specs=pl.BlockSpec((BN, 1), lambda i: (i, 0)),
        out_shape=jax.ShapeDtypeStruct((Npad, 1), jnp.float32),
    )(sp0, sp1, y2, dinv, a2, c2, Wc1, bc1, Wc2, bc2)


def kernel(x, edge_index, edge_attr, W1, b1, g1, be1, W2, b2, g2, be2,
           Wc1, bc1, Wc2, bc2):
    N, F = x.shape
    H = W1.shape[1]
    C1 = Wc1.shape[1]
    E = edge_attr.shape[0]
    Npad = ((N + NS * L - 1) // (NS * L)) * (NS * L)

    C = 80
    row = edge_index[0].astype(jnp.int32).reshape(E // C, C)
    col = edge_index[1].astype(jnp.int32).reshape(E // C, C)
    ew = edge_attr.astype(jnp.float32).reshape(E // C, C)

    deg_call = _make_deg_kernel(E, Npad, C)
    msg_call = _make_msg_kernel(Npad, E, H, C)

    degp = deg_call(col, ew)
    y1, dinv = _tc1(x, W1, degp[0].reshape(Npad, 1), degp[1].reshape(Npad, 1),
                    Npad)

    isq = float(1.0 + 1e-5) ** -0.5
    a1 = (g1 * isq).reshape(1, H)
    c1 = (g1 * isq * b1 + be1).reshape(1, H)
    a2 = (g2 * isq).reshape(1, H)
    c2 = (g2 * isq * b2 + be2).reshape(1, H)

    sp1 = msg_call(y1, row, col, ew)
    y2 = _tc2(sp1[0], sp1[1], y1, dinv, a1, c1, W2)
    sp2 = msg_call(y2, row, col, ew)
    out = _tc3(sp2[0], sp2[1], y2, dinv, a2, c2, Wc1, bc1.reshape(1, C1),
               Wc2, bc2.reshape(1, 1))
    return out.reshape(Npad)[:N]
